# baseline (speedup 1.0000x reference)
import jax
import jax.numpy as jnp
from jax import lax
from jax.experimental import pallas as pl
from jax.experimental.pallas import tpu as pltpu

_BATCH = 64
_SEQ_LEN = 262144
_MAX_DROP_LENGTH = 2048
_WIN = _MAX_DROP_LENGTH + 128
_NG = _BATCH // 8


def _drop_bounds(batch, seq_len):
    rkey = jax.random.key(42)
    k_start, k_len = jax.random.split(rkey)
    drop_start = jax.random.randint(k_start, (batch,), 0, seq_len // 2)
    drop_len = jax.random.randint(k_len, (batch,), 0, _MAX_DROP_LENGTH)
    drop_end = jnp.minimum(drop_start + drop_len, seq_len)
    return drop_start.astype(jnp.int32), drop_end.astype(jnp.int32)


def _fix_kernel(ca_ref, s_in, e_in, ca_in, x_hbm, cp_any, o_hbm,
                wins, sem_in, sem_out):
    del cp_any

    def in_copy(r):
        ca = pl.multiple_of(ca_ref[r], 128)
        g = r // 8
        return pltpu.make_async_copy(
            x_hbm.at[pl.ds(8 * g, 8), pl.ds(ca, _WIN)],
            wins.at[pl.ds(r * 8, 8)], sem_in.at[r])

    def out_copy(r):
        ca = pl.multiple_of(ca_ref[r], 128)
        g = r // 8
        return pltpu.make_async_copy(
            wins.at[pl.ds(r * 8, 8)],
            o_hbm.at[pl.ds(8 * g, 8), pl.ds(ca, _WIN)], sem_out.at[r])

    for r in range(_BATCH):
        in_copy(r).start()
    for r in range(_BATCH):
        in_copy(r).wait()

    col = ca_in[:, 0:1] + lax.broadcasted_iota(jnp.int32, (_BATCH * 8, _WIN), 1)
    mask = (col >= s_in[:, 0:1]) & (col < e_in[:, 0:1])
    wins[...] = jnp.where(mask, jnp.zeros((), wins.dtype), wins[...])

    for r in range(_BATCH):
        out_copy(r).start()
    for r in range(_BATCH):
        out_copy(r).wait()


def kernel(waveform):
    batch, seq_len = waveform.shape
    s, e = _drop_bounds(batch, seq_len)
    ca = (s // 128) * 128

    cag = jnp.broadcast_to(ca.reshape(_NG, 8, 1), (_NG, 8, 8)).reshape(-1)
    sg = jnp.broadcast_to(s.reshape(_NG, 1, 8), (_NG, 8, 8)).reshape(-1)
    eg = jnp.broadcast_to(e.reshape(_NG, 1, 8), (_NG, 8, 8)).reshape(-1)
    s_in = jnp.broadcast_to(sg[:, None], (batch * 8, 128))
    e_in = jnp.broadcast_to(eg[:, None], (batch * 8, 128))
    ca_in = jnp.broadcast_to(cag[:, None], (batch * 8, 128))

    cp = jax.freeze(jax.new_ref(waveform))

    fix = pl.pallas_call(
        _fix_kernel,
        out_shape=jax.ShapeDtypeStruct((batch, seq_len), waveform.dtype),
        grid_spec=pltpu.PrefetchScalarGridSpec(
            num_scalar_prefetch=1,
            grid=(1,),
            in_specs=[
                pl.BlockSpec((batch * 8, 128), lambda i, *_: (0, 0)),
                pl.BlockSpec((batch * 8, 128), lambda i, *_: (0, 0)),
                pl.BlockSpec((batch * 8, 128), lambda i, *_: (0, 0)),
                pl.BlockSpec(memory_space=pl.ANY),
                pl.BlockSpec(memory_space=pl.ANY),
            ],
            out_specs=pl.BlockSpec(memory_space=pl.ANY),
            scratch_shapes=[
                pltpu.VMEM((_BATCH * 8, _WIN), jnp.float32),
                pltpu.SemaphoreType.DMA((_BATCH,)),
                pltpu.SemaphoreType.DMA((_BATCH,)),
            ],
        ),
        input_output_aliases={5: 0},
    )
    return fix(ca, s_in, e_in, ca_in, waveform, cp)


# speedup vs baseline: 1.0752x; 1.0752x over previous
import jax
import jax.numpy as jnp
from jax import lax
from jax.experimental import pallas as pl
from jax.experimental.pallas import tpu as pltpu

_BATCH = 64
_SEQ_LEN = 262144
_MAX_DROP_LENGTH = 2048
_WIN = _MAX_DROP_LENGTH + 128


def _drop_bounds(batch, seq_len):
    rkey = jax.random.key(42)
    k_start, k_len = jax.random.split(rkey)
    drop_start = jax.random.randint(k_start, (batch,), 0, seq_len // 2)
    drop_len = jax.random.randint(k_len, (batch,), 0, _MAX_DROP_LENGTH)
    drop_end = jnp.minimum(drop_start + drop_len, seq_len)
    return drop_start.astype(jnp.int32), drop_end.astype(jnp.int32)


def _fix_kernel(s_ref, e_ref, ca_ref, x_hbm, cp_any, o_hbm,
                scratch, sem_in, sem_out):
    del cp_any

    def in_copy(r):
        ca = pl.multiple_of(ca_ref[r], 128)
        return pltpu.make_async_copy(
            x_hbm.at[r, pl.ds(ca, _WIN)], scratch.at[r], sem_in.at[r])

    def out_copy(r):
        ca = pl.multiple_of(ca_ref[r], 128)
        return pltpu.make_async_copy(
            scratch.at[r], o_hbm.at[r, pl.ds(ca, _WIN)], sem_out.at[r])

    for r in range(_BATCH):
        in_copy(r).start()
    for r in range(_BATCH):
        in_copy(r).wait()
        s = s_ref[r]
        e = e_ref[r]
        col = ca_ref[r] + lax.broadcasted_iota(jnp.int32, (1, _WIN), 1)
        mask = (col >= s) & (col < e)
        v = scratch[r:r + 1, :]
        scratch[r:r + 1, :] = jnp.where(mask, jnp.zeros((), v.dtype), v)
        out_copy(r).start()
    for r in range(_BATCH):
        out_copy(r).wait()


def kernel(waveform):
    batch, seq_len = waveform.shape
    s, e = _drop_bounds(batch, seq_len)
    ca = (s // 128) * 128

    cp = jax.freeze(jax.new_ref(waveform))

    fix = pl.pallas_call(
        _fix_kernel,
        out_shape=jax.ShapeDtypeStruct((batch, seq_len), waveform.dtype),
        grid_spec=pltpu.PrefetchScalarGridSpec(
            num_scalar_prefetch=3,
            grid=(1,),
            in_specs=[
                pl.BlockSpec(memory_space=pl.ANY),
                pl.BlockSpec(memory_space=pl.ANY),
            ],
            out_specs=pl.BlockSpec(memory_space=pl.ANY),
            scratch_shapes=[
                pltpu.VMEM((_BATCH, _WIN), jnp.float32),
                pltpu.SemaphoreType.DMA((_BATCH,)),
                pltpu.SemaphoreType.DMA((_BATCH,)),
            ],
        ),
        input_output_aliases={4: 0},
    )
    return fix(s, e, ca, waveform, cp)
